# Initial kernel scaffold; baseline (speedup 1.0000x reference)
import functools

import jax
import jax.numpy as jnp
from jax import lax
from jax.experimental import pallas as pl
from jax.experimental.pallas import tpu as pltpu
from jax.experimental.pallas import tpu_sc as plsc

B = 1024
S = 200
H = 768
SP = 10

NC, NS = 2, 16
NW = NC * NS
SEQ_PER_W = B // NW
ROWS_PER_W = SEQ_PER_W * S

CH = 40
CHUNKS_PER_SEQ = S // CH
NBUF = 2
NCH = SEQ_PER_W * CHUNKS_PER_SEQ


def _sc_body(xf_hbm, wte_hbm, sp_hbm, out_hbm,
             idx_v, buf0, buf1, sp_v, g0, g1, s0, s1, psem):
    bufs = (buf0, buf1)
    gsems = (g0, g1)
    ssems = (s0, s1)

    wid = lax.axis_index("s") * NC + lax.axis_index("c")
    row_base = wid * ROWS_PER_W

    pltpu.sync_copy(xf_hbm.at[pl.ds(row_base, ROWS_PER_W)], idx_v)
    pltpu.sync_copy(sp_hbm, sp_v)

    def gather(c, b):
        return pltpu.make_async_copy(
            wte_hbm.at[idx_v.at[pl.ds(c * CH, CH)]], bufs[b], gsems[b])

    def scatter(c, b):
        return pltpu.make_async_copy(
            bufs[b], out_hbm.at[pl.ds(row_base + c * CH, CH)], ssems[b])

    for b in range(NBUF):
        gather(b, b).start()

    def step(g, carry):
        for b in range(NBUF):
            c = g * NBUF + b
            gather(c, b).wait()
            scatter(c, b).start()
            scatter(c, b).wait()

            @pl.when(lax.rem(c, CHUNKS_PER_SEQ) == 0)
            def _():
                pltpu.make_async_copy(
                    sp_v, out_hbm.at[pl.ds(row_base + c * CH, SP)], psem
                ).start()

            @pl.when(c + NBUF < NCH)
            def _():
                gather(c + NBUF, b).start()
        return carry

    lax.fori_loop(0, NCH // NBUF, step, 0)

    def drain(i, carry):
        pltpu.make_async_copy(
            sp_v, out_hbm.at[pl.ds(row_base, SP)], psem).wait()
        return carry
    lax.fori_loop(0, SEQ_PER_W, drain, 0)


@jax.jit
def kernel(x, wte, soft_prompt):
    xf = x.reshape(B * S)
    mesh = plsc.VectorSubcoreMesh(core_axis_name="c", subcore_axis_name="s")
    k = functools.partial(
        pl.kernel,
        mesh=mesh,
        out_type=jax.ShapeDtypeStruct((B * S, H), jnp.float32),
        scratch_types=[
            pltpu.VMEM((ROWS_PER_W,), jnp.int32),
            pltpu.VMEM((CH, H), jnp.float32),
            pltpu.VMEM((CH, H), jnp.float32),
            pltpu.VMEM((SP, H), jnp.float32),
            pltpu.SemaphoreType.DMA,
            pltpu.SemaphoreType.DMA,
            pltpu.SemaphoreType.DMA,
            pltpu.SemaphoreType.DMA,
            pltpu.SemaphoreType.DMA,
        ],
    )(_sc_body)
    out = k(xf, wte, soft_prompt)
    return out.reshape(B, S, H)


# speedup vs baseline: 2.6550x; 2.6550x over previous
import functools

import jax
import jax.numpy as jnp
from jax import lax
from jax.experimental import pallas as pl
from jax.experimental.pallas import tpu as pltpu
from jax.experimental.pallas import tpu_sc as plsc

B = 1024
S = 200
H = 768
SP = 10
L = 16

NC, NS = 2, 16
NW = NC * NS
SEQ_PER_W = B // NW
ROWS_PER_W = SEQ_PER_W * S

CH = 40
CHUNKS_PER_SEQ = S // CH
NBUF = 2
NCH = SEQ_PER_W * CHUNKS_PER_SEQ


def _sc_body(xf_hbm, wte_hbm, sp_hbm, out_hbm,
             idx_v, buf0, buf1, sp_v, g0, g1, s0, s1, psem):
    bufs = (buf0, buf1)
    gsems = (g0, g1)
    ssems = (s0, s1)

    wid = lax.axis_index("s") * NC + lax.axis_index("c")
    row_base = wid * ROWS_PER_W

    pltpu.sync_copy(xf_hbm.at[pl.ds(row_base, ROWS_PER_W)], idx_v)
    pltpu.sync_copy(sp_hbm, sp_v)

    def gather(c, b):
        return pltpu.make_async_copy(
            wte_hbm.at[idx_v.at[pl.ds(c * CH, CH)]], bufs[b], gsems[b])

    def scatter_full(c, b):
        return pltpu.make_async_copy(
            bufs[b], out_hbm.at[pl.ds(row_base + c * CH, CH)], ssems[b])

    def scatter_tail(c, b):
        return pltpu.make_async_copy(
            bufs[b].at[pl.ds(8, CH - 8)],
            out_hbm.at[pl.ds(row_base + c * CH + 8, CH - 8)], ssems[b])

    for b in range(NBUF):
        gather(b, b).start()

    def step(g, carry):
        for b in range(NBUF):
            c = g * NBUF + b
            gather(c, b).wait()
            seq_start = lax.rem(c, CHUNKS_PER_SEQ) == 0

            @pl.when(seq_start)
            def _():
                for i in (8, 9):
                    for j in range(H // L):
                        bufs[b][i, pl.ds(j * L, L)] = sp_v[i, pl.ds(j * L, L)]
                scatter_tail(c, b).start()
                pltpu.make_async_copy(
                    sp_v.at[pl.ds(0, 8)],
                    out_hbm.at[pl.ds(row_base + c * CH, 8)], psem).start()
                scatter_tail(c, b).wait()

            @pl.when(jnp.logical_not(seq_start))
            def _():
                scatter_full(c, b).start()
                scatter_full(c, b).wait()

            @pl.when(c + NBUF < NCH)
            def _():
                gather(c + NBUF, b).start()
        return carry

    lax.fori_loop(0, NCH // NBUF, step, 0)

    def drain(i, carry):
        pltpu.make_async_copy(
            sp_v.at[pl.ds(0, 8)],
            out_hbm.at[pl.ds(row_base, 8)], psem).wait()
        return carry
    lax.fori_loop(0, SEQ_PER_W, drain, 0)


@jax.jit
def kernel(x, wte, soft_prompt):
    xf = x.reshape(B * S)
    sp16 = jnp.zeros((16, H), jnp.float32).at[:SP].set(soft_prompt)
    mesh = plsc.VectorSubcoreMesh(core_axis_name="c", subcore_axis_name="s")
    k = functools.partial(
        pl.kernel,
        mesh=mesh,
        out_type=jax.ShapeDtypeStruct((B * S, H), jnp.float32),
        scratch_types=[
            pltpu.VMEM((ROWS_PER_W,), jnp.int32),
            pltpu.VMEM((CH, H), jnp.float32),
            pltpu.VMEM((CH, H), jnp.float32),
            pltpu.VMEM((16, H), jnp.float32),
            pltpu.SemaphoreType.DMA,
            pltpu.SemaphoreType.DMA,
            pltpu.SemaphoreType.DMA,
            pltpu.SemaphoreType.DMA,
            pltpu.SemaphoreType.DMA,
        ],
    )(_sc_body)
    out = k(xf, wte, sp16)
    return out.reshape(B, S, H)


# speedup vs baseline: 2.6605x; 1.0021x over previous
import functools

import jax
import jax.numpy as jnp
from jax import lax
from jax.experimental import pallas as pl
from jax.experimental.pallas import tpu as pltpu
from jax.experimental.pallas import tpu_sc as plsc

B = 1024
S = 200
H = 768
SP = 10
L = 16

NC, NS = 2, 16
NW = NC * NS
SEQ_PER_W = B // NW
ROWS_PER_W = SEQ_PER_W * S

CH = 40
CHUNKS_PER_SEQ = S // CH
NBUF = 3
NCH = SEQ_PER_W * CHUNKS_PER_SEQ


def _sc_body(xf_hbm, wte_hbm, sp_hbm, out_hbm,
             idx_v, buf0, buf1, buf2, sp_v, g0, g1, g2, s0, s1, s2, psem):
    bufs = (buf0, buf1, buf2)
    gsems = (g0, g1, g2)
    ssems = (s0, s1, s2)

    wid = lax.axis_index("s") * NC + lax.axis_index("c")
    row_base = wid * ROWS_PER_W

    pltpu.sync_copy(xf_hbm.at[pl.ds(row_base, ROWS_PER_W)], idx_v)
    pltpu.sync_copy(sp_hbm, sp_v)

    def gather(c, b):
        return pltpu.make_async_copy(
            wte_hbm.at[idx_v.at[pl.ds(c * CH, CH)]], bufs[b], gsems[b])

    def scatter_full(c, b):
        return pltpu.make_async_copy(
            bufs[b], out_hbm.at[pl.ds(row_base + c * CH, CH)], ssems[b])

    def scatter_tail(c, b):
        return pltpu.make_async_copy(
            bufs[b].at[pl.ds(8, CH - 8)],
            out_hbm.at[pl.ds(row_base + c * CH + 8, CH - 8)], ssems[b])

    def produce(c, b):
        gather(c, b).wait()
        seq_start = lax.rem(c, CHUNKS_PER_SEQ) == 0

        @pl.when(seq_start)
        def _():
            for i in (8, 9):
                for j in range(H // L):
                    bufs[b][i, pl.ds(j * L, L)] = sp_v[i, pl.ds(j * L, L)]
            scatter_tail(c, b).start()
            pltpu.make_async_copy(
                sp_v.at[pl.ds(0, 8)],
                out_hbm.at[pl.ds(row_base + c * CH, 8)], psem).start()

        @pl.when(jnp.logical_not(seq_start))
        def _():
            scatter_full(c, b).start()

    def scatter_wait(c, b):
        @pl.when(lax.rem(c, CHUNKS_PER_SEQ) == 0)
        def _():
            scatter_tail(c, b).wait()

        @pl.when(lax.rem(c, CHUNKS_PER_SEQ) != 0)
        def _():
            scatter_full(c, b).wait()

    gather(0, 0).start()
    gather(1, 1).start()
    produce(0, 0)
    gather(2, 2).start()

    def step(g, carry):
        for boff in range(NBUF):
            c = 1 + g * NBUF + boff
            b = (1 + boff) % NBUF
            produce(c, b)
            scatter_wait(c - 1, (b + NBUF - 1) % NBUF)

            @pl.when(c + 2 < NCH)
            def _():
                gather(c + 2, (b + 2) % NBUF).start()
        return carry

    lax.fori_loop(0, (NCH - 1) // NBUF, step, 0)
    scatter_wait(NCH - 1, (NCH - 1) % NBUF)

    def drain(i, carry):
        pltpu.make_async_copy(
            sp_v.at[pl.ds(0, 8)],
            out_hbm.at[pl.ds(row_base, 8)], psem).wait()
        return carry
    lax.fori_loop(0, SEQ_PER_W, drain, 0)


@jax.jit
def kernel(x, wte, soft_prompt):
    xf = x.reshape(B * S)
    sp16 = jnp.zeros((16, H), jnp.float32).at[:SP].set(soft_prompt)
    mesh = plsc.VectorSubcoreMesh(core_axis_name="c", subcore_axis_name="s")
    k = functools.partial(
        pl.kernel,
        mesh=mesh,
        out_type=jax.ShapeDtypeStruct((B * S, H), jnp.float32),
        scratch_types=[
            pltpu.VMEM((ROWS_PER_W,), jnp.int32),
            pltpu.VMEM((CH, H), jnp.float32),
            pltpu.VMEM((CH, H), jnp.float32),
            pltpu.VMEM((CH, H), jnp.float32),
            pltpu.VMEM((16, H), jnp.float32),
            pltpu.SemaphoreType.DMA,
            pltpu.SemaphoreType.DMA,
            pltpu.SemaphoreType.DMA,
            pltpu.SemaphoreType.DMA,
            pltpu.SemaphoreType.DMA,
            pltpu.SemaphoreType.DMA,
            pltpu.SemaphoreType.DMA,
        ],
    )(_sc_body)
    out = k(xf, wte, sp16)
    return out.reshape(B, S, H)


# speedup vs baseline: 2.7005x; 1.0150x over previous
import functools

import jax
import jax.numpy as jnp
from jax import lax
from jax.experimental import pallas as pl
from jax.experimental.pallas import tpu as pltpu
from jax.experimental.pallas import tpu_sc as plsc

B = 1024
S = 200
H = 768
SP = 10
L = 16

NC, NS = 2, 16
NW = NC * NS
SEQ_PER_W = B // NW
ROWS_PER_W = SEQ_PER_W * S

CH = 40
CHUNKS_PER_SEQ = S // CH
NBUF = 3
NCH = SEQ_PER_W * CHUNKS_PER_SEQ


def _sc_body(xf_hbm, wte_hbm, sp_hbm, out_hbm,
             idx_v, buf0, buf1, buf2, sp_v, g0, g1, g2, s0, s1, s2, psem):
    bufs = (buf0, buf1, buf2)
    gsems = (g0, g1, g2)
    ssems = (s0, s1, s2)

    wid = lax.axis_index("s") * NC + lax.axis_index("c")
    row_base = wid * ROWS_PER_W

    pltpu.sync_copy(xf_hbm.at[pl.ds(row_base, ROWS_PER_W)], idx_v)
    pltpu.sync_copy(sp_hbm, sp_v)

    def gather_full(c, b):
        return pltpu.make_async_copy(
            wte_hbm.at[idx_v.at[pl.ds(c * CH, CH)]], bufs[b], gsems[b])

    def gather_head(c, b):
        return pltpu.make_async_copy(
            wte_hbm.at[idx_v.at[pl.ds(c * CH + 8, CH - 8)]],
            bufs[b].at[pl.ds(8, CH - 8)], gsems[b])

    def scatter_full(c, b):
        return pltpu.make_async_copy(
            bufs[b], out_hbm.at[pl.ds(row_base + c * CH, CH)], ssems[b])

    def scatter_tail(c, b):
        return pltpu.make_async_copy(
            bufs[b].at[pl.ds(8, CH - 8)],
            out_hbm.at[pl.ds(row_base + c * CH + 8, CH - 8)], ssems[b])

    def on_head(c, head_fn, rest_fn):
        if isinstance(c, int):
            (head_fn if c % CHUNKS_PER_SEQ == 0 else rest_fn)()
        else:
            rem = lax.rem(c, CHUNKS_PER_SEQ)
            pl.when(rem == 0)(head_fn)
            pl.when(rem != 0)(rest_fn)

    def gather_start(c, b):
        on_head(c, lambda: gather_head(c, b).start(),
                lambda: gather_full(c, b).start())

    def produce(c, b):
        def head():
            gather_head(c, b).wait()
            for i in (8, 9):
                for j in range(H // L):
                    bufs[b][i, pl.ds(j * L, L)] = sp_v[i, pl.ds(j * L, L)]
            scatter_tail(c, b).start()
            pltpu.make_async_copy(
                sp_v.at[pl.ds(0, 8)],
                out_hbm.at[pl.ds(row_base + c * CH, 8)], psem).start()

        def rest():
            gather_full(c, b).wait()
            scatter_full(c, b).start()

        on_head(c, head, rest)

    def scatter_wait(c, b):
        on_head(c, lambda: scatter_tail(c, b).wait(),
                lambda: scatter_full(c, b).wait())

    gather_start(0, 0)
    gather_start(1, 1)
    produce(0, 0)
    gather_start(2, 2)

    def step(g, carry):
        for boff in range(NBUF):
            c = 1 + g * NBUF + boff
            b = (1 + boff) % NBUF
            produce(c, b)
            scatter_wait(c - 1, (b + NBUF - 1) % NBUF)

            @pl.when(c + 2 < NCH)
            def _():
                gather_start(c + 2, (b + 2) % NBUF)
        return carry

    lax.fori_loop(0, (NCH - 1) // NBUF, step, 0)
    scatter_wait(NCH - 1, (NCH - 1) % NBUF)

    def drain(i, carry):
        pltpu.make_async_copy(
            sp_v.at[pl.ds(0, 8)],
            out_hbm.at[pl.ds(row_base, 8)], psem).wait()
        return carry
    lax.fori_loop(0, SEQ_PER_W, drain, 0)


@jax.jit
def kernel(x, wte, soft_prompt):
    xf = x.reshape(B * S)
    sp16 = jnp.zeros((16, H), jnp.float32).at[:SP].set(soft_prompt)
    mesh = plsc.VectorSubcoreMesh(core_axis_name="c", subcore_axis_name="s")
    k = functools.partial(
        pl.kernel,
        mesh=mesh,
        out_type=jax.ShapeDtypeStruct((B * S, H), jnp.float32),
        scratch_types=[
            pltpu.VMEM((ROWS_PER_W,), jnp.int32),
            pltpu.VMEM((CH, H), jnp.float32),
            pltpu.VMEM((CH, H), jnp.float32),
            pltpu.VMEM((CH, H), jnp.float32),
            pltpu.VMEM((16, H), jnp.float32),
            pltpu.SemaphoreType.DMA,
            pltpu.SemaphoreType.DMA,
            pltpu.SemaphoreType.DMA,
            pltpu.SemaphoreType.DMA,
            pltpu.SemaphoreType.DMA,
            pltpu.SemaphoreType.DMA,
            pltpu.SemaphoreType.DMA,
        ],
    )(_sc_body)
    out = k(xf, wte, sp16)
    return out.reshape(B, S, H)
